# baseline (speedup 1.0000x reference)
import functools

import jax
import jax.numpy as jnp
from jax.experimental import pallas as pl
from jax.experimental.pallas import tpu as pltpu


_SQRT_HALF = 0.7071067811865476


def _gelu(x):
    return 0.5 * x * (1.0 + jax.lax.erf(x * _SQRT_HALF))


def _fold_w(w):
    cin, cout = w.shape[0], w.shape[1]
    wk = jnp.transpose(w, (0, 2, 3, 1)).reshape(cin, 4 * cout)
    return wk.astype(jnp.bfloat16)


def _row_view_bf16(x):
    n, c, h, w = x.shape
    return jnp.transpose(x, (0, 2, 3, 1)).reshape(n * h * w, c).astype(jnp.bfloat16)


def _deconv4x_kernel(x_ref, w1_ref, s1_ref, t1_ref, w2_ref, t2_ref, o_ref, *,
                     c, th, w):
    y1 = jnp.dot(x_ref[...], w1_ref[...], preferred_element_type=jnp.float32)
    y1 = _gelu(y1 * s1_ref[...] + t1_ref[...]).astype(jnp.bfloat16)
    t2 = t2_ref[...]
    for j in range(4):
        dh1, dw1 = j // 2, j % 2
        z = jnp.dot(y1[:, j * c:(j + 1) * c], w2_ref[...],
                    preferred_element_type=jnp.float32) + t2
        for k in range(4):
            dh2, dw2 = k // 2, k % 2
            a, b = 2 * dh1 + dh2, 2 * dw1 + dw2
            piece = z[:, k * c:(k + 1) * c].reshape(th, w, c)
            o_ref[0, :, a, :, b, 0, :] = piece.astype(o_ref.dtype)


def _deconv4x_nhwc(x2d, w1, b1, gamma, beta, mean, var, w2, b2, *, n, h, w,
                   eps=1e-5):
    m, cin = x2d.shape
    c = w1.shape[1]
    wk1 = _fold_w(w1)
    wk2 = jnp.transpose(w2, (0, 2, 3, 1)).reshape(cin, 4 * c).astype(jnp.bfloat16)
    s = (gamma / jnp.sqrt(var + eps)).astype(jnp.float32)
    t = b1.astype(jnp.float32) * s + (beta - mean * s).astype(jnp.float32)
    s1 = jnp.tile(s, 4).reshape(1, 4 * c)
    t1 = jnp.tile(t, 4).reshape(1, 4 * c)
    t2 = jnp.tile(b2.astype(jnp.float32), 4).reshape(1, 4 * c)
    th = 16
    tm = th * w
    steps_per_n = h // th
    kernel_fn = functools.partial(_deconv4x_kernel, c=c, th=th, w=w)
    out = pl.pallas_call(
        kernel_fn,
        out_shape=jax.ShapeDtypeStruct((n, h, 4, w, 4, 1, c), jnp.float32),
        grid=(m // tm,),
        in_specs=[
            pl.BlockSpec((tm, cin), lambda i: (i, 0)),
            pl.BlockSpec((cin, 4 * c), lambda i: (0, 0)),
            pl.BlockSpec((1, 4 * c), lambda i: (0, 0)),
            pl.BlockSpec((1, 4 * c), lambda i: (0, 0)),
            pl.BlockSpec((cin, 4 * c), lambda i: (0, 0)),
            pl.BlockSpec((1, 4 * c), lambda i: (0, 0)),
        ],
        out_specs=pl.BlockSpec(
            (1, th, 4, w, 4, 1, c),
            lambda i, s=steps_per_n: (i // s, i % s, 0, 0, 0, 0, 0)),
        compiler_params=pltpu.CompilerParams(
            dimension_semantics=("parallel",)),
    )(x2d, wk1, s1, t1, wk2, t2)
    return out


def _deconv2x_kernel(x_ref, w_ref, b_ref, o_ref, *, c, th, w):
    z = jnp.dot(x_ref[...], w_ref[...],
                preferred_element_type=jnp.float32) + b_ref[...]
    for k in range(4):
        dh, dw = k // 2, k % 2
        piece = z[:, k * c:(k + 1) * c].reshape(th, w, c)
        o_ref[0, :, dh, :, dw, 0, :] = piece.astype(o_ref.dtype)


def _deconv2x_nhwc(x2d, w2, b, *, n, h, w):
    m, cin = x2d.shape
    c = w2.shape[1]
    wk = jnp.transpose(w2, (0, 2, 3, 1)).reshape(cin, 4 * c).astype(jnp.bfloat16)
    bias = jnp.tile(b.astype(jnp.float32), 4).reshape(1, 4 * c)
    th = 32
    tm = th * w
    steps_per_n = h // th
    return pl.pallas_call(
        functools.partial(_deconv2x_kernel, c=c, th=th, w=w),
        out_shape=jax.ShapeDtypeStruct((n, h, 2, w, 2, 1, c), jnp.float32),
        grid=(m // tm,),
        in_specs=[
            pl.BlockSpec((tm, cin), lambda i: (i, 0)),
            pl.BlockSpec((cin, 4 * c), lambda i: (0, 0)),
            pl.BlockSpec((1, 4 * c), lambda i: (0, 0)),
        ],
        out_specs=pl.BlockSpec(
            (1, th, 2, w, 2, 1, c),
            lambda i, s=steps_per_n: (i // s, i % s, 0, 0, 0, 0, 0)),
        compiler_params=pltpu.CompilerParams(
            dimension_semantics=("parallel",)),
    )(x2d, wk, bias)


def _maxpool_kernel(x_ref, o_ref, *, c):
    x = x_ref[...]
    hm = jnp.maximum(x[:, 0], x[:, 1])
    o_ref[...] = jnp.maximum(hm[:, :, :c], hm[:, :, c:])


def _maxpool2x2(x):
    n, c, h, w = x.shape
    ho, wo = h // 2, w // 2
    xh = jnp.transpose(x, (0, 2, 3, 1)).reshape(n * ho, 2, wo, 2 * c)
    rows = n * ho
    tb = rows // 2
    out = pl.pallas_call(
        functools.partial(_maxpool_kernel, c=c),
        out_shape=jax.ShapeDtypeStruct((rows, wo, c), x.dtype),
        grid=(rows // tb,),
        in_specs=[pl.BlockSpec((tb, 2, wo, 2 * c), lambda i: (i, 0, 0, 0))],
        out_specs=pl.BlockSpec((tb, wo, c), lambda i: (i, 0, 0)),
        compiler_params=pltpu.CompilerParams(
            dimension_semantics=("parallel",)),
    )(xh)
    return jnp.transpose(out.reshape(n, ho, wo, c), (0, 3, 1, 2))


def kernel(x0, x1, x2, x3, p0_w1, p0_b1, p0_gamma, p0_beta, p0_mean, p0_var,
           p0_w2, p0_b2, p1_w, p1_b):
    n, c, h, w = x0.shape

    y0 = _deconv4x_nhwc(_row_view_bf16(x0), p0_w1, p0_b1, p0_gamma, p0_beta,
                        p0_mean, p0_var, p0_w2, p0_b2, n=n, h=h, w=w)
    out0 = jnp.transpose(y0.reshape(n, 4 * h, 4 * w, c), (0, 3, 1, 2))

    y1 = _deconv2x_nhwc(_row_view_bf16(x1), p1_w, p1_b, n=n, h=h, w=w)
    out1 = jnp.transpose(y1.reshape(n, 2 * h, 2 * w, c), (0, 3, 1, 2))

    out2 = x2

    out3 = _maxpool2x2(x3)

    return (out0, out1, out2, out3)


# speedup vs baseline: 2.2561x; 2.2561x over previous
import functools

import jax
import jax.numpy as jnp
from jax.experimental import pallas as pl
from jax.experimental.pallas import tpu as pltpu


_SQRT_HALF = 0.7071067811865476


def _gelu(x):
    return 0.5 * x * (1.0 + jax.lax.erf(x * _SQRT_HALF))


def _fold_w(w):
    cin, cout = w.shape[0], w.shape[1]
    wk = jnp.transpose(w, (0, 2, 3, 1)).reshape(cin, 4 * cout)
    return wk.astype(jnp.bfloat16)


def _row_view_bf16(x):
    n, c, h, w = x.shape
    return jnp.transpose(x, (0, 2, 3, 1)).reshape(n * h * w, c).astype(jnp.bfloat16)


def _deconv4x_kernel(x_ref, w1_ref, s1_ref, t1_ref, w2_ref, t2_ref, o_ref, *,
                     c, th, w):
    y1 = jnp.dot(x_ref[...], w1_ref[...], preferred_element_type=jnp.float32)
    y1 = _gelu(y1 * s1_ref[...] + t1_ref[...]).astype(jnp.bfloat16)
    t2 = t2_ref[...]
    for j in range(4):
        dh1, dw1 = j // 2, j % 2
        z = jnp.dot(y1[:, j * c:(j + 1) * c], w2_ref[...],
                    preferred_element_type=jnp.float32) + t2
        for k in range(4):
            dh2, dw2 = k // 2, k % 2
            a, b = 2 * dh1 + dh2, 2 * dw1 + dw2
            piece = z[:, k * c:(k + 1) * c].reshape(th, w, c)
            o_ref[0, :, a, :, b * c:(b + 1) * c] = piece.astype(o_ref.dtype)


def _deconv4x_nhwc(x2d, w1, b1, gamma, beta, mean, var, w2, b2, *, n, h, w,
                   eps=1e-5):
    m, cin = x2d.shape
    c = w1.shape[1]
    wk1 = _fold_w(w1)
    wk2 = jnp.transpose(w2, (0, 2, 3, 1)).reshape(cin, 4 * c).astype(jnp.bfloat16)
    s = (gamma / jnp.sqrt(var + eps)).astype(jnp.float32)
    t = b1.astype(jnp.float32) * s + (beta - mean * s).astype(jnp.float32)
    s1 = jnp.tile(s, 4).reshape(1, 4 * c)
    t1 = jnp.tile(t, 4).reshape(1, 4 * c)
    t2 = jnp.tile(b2.astype(jnp.float32), 4).reshape(1, 4 * c)
    th = 16
    tm = th * w
    steps_per_n = h // th
    kernel_fn = functools.partial(_deconv4x_kernel, c=c, th=th, w=w)
    out = pl.pallas_call(
        kernel_fn,
        out_shape=jax.ShapeDtypeStruct((n, h, 4, w, 4 * c), jnp.float32),
        grid=(m // tm,),
        in_specs=[
            pl.BlockSpec((tm, cin), lambda i: (i, 0)),
            pl.BlockSpec((cin, 4 * c), lambda i: (0, 0)),
            pl.BlockSpec((1, 4 * c), lambda i: (0, 0)),
            pl.BlockSpec((1, 4 * c), lambda i: (0, 0)),
            pl.BlockSpec((cin, 4 * c), lambda i: (0, 0)),
            pl.BlockSpec((1, 4 * c), lambda i: (0, 0)),
        ],
        out_specs=pl.BlockSpec(
            (1, th, 4, w, 4 * c),
            lambda i, s=steps_per_n: (i // s, i % s, 0, 0, 0)),
        compiler_params=pltpu.CompilerParams(
            dimension_semantics=("parallel",)),
    )(x2d, wk1, s1, t1, wk2, t2)
    return out


def _deconv2x_kernel(x_ref, w_ref, b_ref, o_ref, *, c, th, w):
    z = jnp.dot(x_ref[...], w_ref[...],
                preferred_element_type=jnp.float32) + b_ref[...]
    for k in range(4):
        dh, dw = k // 2, k % 2
        piece = z[:, k * c:(k + 1) * c].reshape(th, w, c)
        o_ref[0, :, dh, :, dw * c:(dw + 1) * c] = piece.astype(o_ref.dtype)


def _deconv2x_nhwc(x2d, w2, b, *, n, h, w):
    m, cin = x2d.shape
    c = w2.shape[1]
    wk = jnp.transpose(w2, (0, 2, 3, 1)).reshape(cin, 4 * c).astype(jnp.bfloat16)
    bias = jnp.tile(b.astype(jnp.float32), 4).reshape(1, 4 * c)
    th = 32
    tm = th * w
    steps_per_n = h // th
    return pl.pallas_call(
        functools.partial(_deconv2x_kernel, c=c, th=th, w=w),
        out_shape=jax.ShapeDtypeStruct((n, h, 2, w, 2 * c), jnp.float32),
        grid=(m // tm,),
        in_specs=[
            pl.BlockSpec((tm, cin), lambda i: (i, 0)),
            pl.BlockSpec((cin, 4 * c), lambda i: (0, 0)),
            pl.BlockSpec((1, 4 * c), lambda i: (0, 0)),
        ],
        out_specs=pl.BlockSpec(
            (1, th, 2, w, 2 * c),
            lambda i, s=steps_per_n: (i // s, i % s, 0, 0, 0)),
        compiler_params=pltpu.CompilerParams(
            dimension_semantics=("parallel",)),
    )(x2d, wk, bias)


def _maxpool_kernel(x_ref, o_ref, *, c):
    x = x_ref[...]
    hm = jnp.maximum(x[:, 0], x[:, 1])
    o_ref[...] = jnp.maximum(hm[:, :, :c], hm[:, :, c:])


def _maxpool2x2(x):
    n, c, h, w = x.shape
    ho, wo = h // 2, w // 2
    xh = jnp.transpose(x, (0, 2, 3, 1)).reshape(n * ho, 2, wo, 2 * c)
    rows = n * ho
    tb = rows // 2
    out = pl.pallas_call(
        functools.partial(_maxpool_kernel, c=c),
        out_shape=jax.ShapeDtypeStruct((rows, wo, c), x.dtype),
        grid=(rows // tb,),
        in_specs=[pl.BlockSpec((tb, 2, wo, 2 * c), lambda i: (i, 0, 0, 0))],
        out_specs=pl.BlockSpec((tb, wo, c), lambda i: (i, 0, 0)),
        compiler_params=pltpu.CompilerParams(
            dimension_semantics=("parallel",)),
    )(xh)
    return jnp.transpose(out.reshape(n, ho, wo, c), (0, 3, 1, 2))


def kernel(x0, x1, x2, x3, p0_w1, p0_b1, p0_gamma, p0_beta, p0_mean, p0_var,
           p0_w2, p0_b2, p1_w, p1_b):
    n, c, h, w = x0.shape

    y0 = _deconv4x_nhwc(_row_view_bf16(x0), p0_w1, p0_b1, p0_gamma, p0_beta,
                        p0_mean, p0_var, p0_w2, p0_b2, n=n, h=h, w=w)
    out0 = jnp.transpose(y0.reshape(n, 4 * h, 4 * w, c), (0, 3, 1, 2))

    y1 = _deconv2x_nhwc(_row_view_bf16(x1), p1_w, p1_b, n=n, h=h, w=w)
    out1 = jnp.transpose(y1.reshape(n, 2 * h, 2 * w, c), (0, 3, 1, 2))

    out2 = x2

    out3 = _maxpool2x2(x3)

    return (out0, out1, out2, out3)


# speedup vs baseline: 2.8658x; 1.2703x over previous
import functools

import jax
import jax.numpy as jnp
from jax.experimental import pallas as pl
from jax.experimental.pallas import tpu as pltpu


_SQRT_HALF = 0.7071067811865476


def _gelu(x):
    return 0.5 * x * (1.0 + jax.lax.erf(x * _SQRT_HALF))


def _fold_w(w):
    cin, cout = w.shape[0], w.shape[1]
    wk = jnp.transpose(w, (0, 2, 3, 1)).reshape(cin, 4 * cout)
    return wk.astype(jnp.bfloat16)


def _row_view_bf16(x):
    n, c, h, w = x.shape
    return jnp.transpose(x, (0, 2, 3, 1)).reshape(n * h * w, c).astype(jnp.bfloat16)


def _deconv4x_kernel(x_ref, w1_ref, s1_ref, t1_ref, w2_ref, t2_ref, o_ref, *,
                     c, th, w):
    y1 = jnp.dot(x_ref[...], w1_ref[...], preferred_element_type=jnp.float32)
    y1 = _gelu(y1 * s1_ref[...] + t1_ref[...]).astype(jnp.bfloat16)
    t2 = t2_ref[...]
    pieces = {}
    for j in range(4):
        dh1, dw1 = j // 2, j % 2
        z = jnp.dot(y1[:, j * c:(j + 1) * c], w2_ref[...],
                    preferred_element_type=jnp.float32) + t2
        for k in range(4):
            dh2, dw2 = k // 2, k % 2
            a, b = 2 * dh1 + dh2, 2 * dw1 + dw2
            pieces[(a, b)] = z[:, k * c:(k + 1) * c].reshape(th, w, c)
    rows_a = []
    for a in range(4):
        qa = jnp.stack([pieces[(a, b)] for b in range(4)], axis=2)
        rows_a.append(qa.reshape(th, 4 * w, c))
    blk = jnp.stack(rows_a, axis=1).reshape(4 * th, 4 * w, c)
    o_ref[0] = blk.astype(o_ref.dtype)


def _deconv4x_nhwc(x2d, w1, b1, gamma, beta, mean, var, w2, b2, *, n, h, w,
                   eps=1e-5):
    m, cin = x2d.shape
    c = w1.shape[1]
    wk1 = _fold_w(w1)
    wk2 = jnp.transpose(w2, (0, 2, 3, 1)).reshape(cin, 4 * c).astype(jnp.bfloat16)
    s = (gamma / jnp.sqrt(var + eps)).astype(jnp.float32)
    t = b1.astype(jnp.float32) * s + (beta - mean * s).astype(jnp.float32)
    s1 = jnp.tile(s, 4).reshape(1, 4 * c)
    t1 = jnp.tile(t, 4).reshape(1, 4 * c)
    t2 = jnp.tile(b2.astype(jnp.float32), 4).reshape(1, 4 * c)
    th = 16
    tm = th * w
    steps_per_n = h // th
    kernel_fn = functools.partial(_deconv4x_kernel, c=c, th=th, w=w)
    out = pl.pallas_call(
        kernel_fn,
        out_shape=jax.ShapeDtypeStruct((n, 4 * h, 4 * w, c), jnp.float32),
        grid=(m // tm,),
        in_specs=[
            pl.BlockSpec((tm, cin), lambda i: (i, 0)),
            pl.BlockSpec((cin, 4 * c), lambda i: (0, 0)),
            pl.BlockSpec((1, 4 * c), lambda i: (0, 0)),
            pl.BlockSpec((1, 4 * c), lambda i: (0, 0)),
            pl.BlockSpec((cin, 4 * c), lambda i: (0, 0)),
            pl.BlockSpec((1, 4 * c), lambda i: (0, 0)),
        ],
        out_specs=pl.BlockSpec(
            (1, 4 * th, 4 * w, c),
            lambda i, s=steps_per_n: (i // s, i % s, 0, 0)),
        compiler_params=pltpu.CompilerParams(
            dimension_semantics=("parallel",)),
    )(x2d, wk1, s1, t1, wk2, t2)
    return out


def _deconv2x_kernel(x_ref, w_ref, b_ref, o_ref, *, c, th, w):
    z = jnp.dot(x_ref[...], w_ref[...],
                preferred_element_type=jnp.float32) + b_ref[...]
    pieces = {}
    for k in range(4):
        dh, dw = k // 2, k % 2
        pieces[(dh, dw)] = z[:, k * c:(k + 1) * c].reshape(th, w, c)
    rows = []
    for dh in range(2):
        q = jnp.stack([pieces[(dh, 0)], pieces[(dh, 1)]], axis=2)
        rows.append(q.reshape(th, 2 * w, c))
    blk = jnp.stack(rows, axis=1).reshape(2 * th, 2 * w, c)
    o_ref[0] = blk.astype(o_ref.dtype)


def _deconv2x_nhwc(x2d, w2, b, *, n, h, w):
    m, cin = x2d.shape
    c = w2.shape[1]
    wk = jnp.transpose(w2, (0, 2, 3, 1)).reshape(cin, 4 * c).astype(jnp.bfloat16)
    bias = jnp.tile(b.astype(jnp.float32), 4).reshape(1, 4 * c)
    th = 32
    tm = th * w
    steps_per_n = h // th
    return pl.pallas_call(
        functools.partial(_deconv2x_kernel, c=c, th=th, w=w),
        out_shape=jax.ShapeDtypeStruct((n, 2 * h, 2 * w, c), jnp.float32),
        grid=(m // tm,),
        in_specs=[
            pl.BlockSpec((tm, cin), lambda i: (i, 0)),
            pl.BlockSpec((cin, 4 * c), lambda i: (0, 0)),
            pl.BlockSpec((1, 4 * c), lambda i: (0, 0)),
        ],
        out_specs=pl.BlockSpec(
            (1, 2 * th, 2 * w, c),
            lambda i, s=steps_per_n: (i // s, i % s, 0, 0)),
        compiler_params=pltpu.CompilerParams(
            dimension_semantics=("parallel",)),
    )(x2d, wk, bias)


def _maxpool_kernel(x_ref, o_ref, *, c):
    x = x_ref[...]
    hm = jnp.maximum(x[:, 0], x[:, 1])
    o_ref[...] = jnp.maximum(hm[:, :, :c], hm[:, :, c:])


def _maxpool2x2(x):
    n, c, h, w = x.shape
    ho, wo = h // 2, w // 2
    xh = jnp.transpose(x, (0, 2, 3, 1)).reshape(n * ho, 2, wo, 2 * c)
    rows = n * ho
    tb = rows // 2
    out = pl.pallas_call(
        functools.partial(_maxpool_kernel, c=c),
        out_shape=jax.ShapeDtypeStruct((rows, wo, c), x.dtype),
        grid=(rows // tb,),
        in_specs=[pl.BlockSpec((tb, 2, wo, 2 * c), lambda i: (i, 0, 0, 0))],
        out_specs=pl.BlockSpec((tb, wo, c), lambda i: (i, 0, 0)),
        compiler_params=pltpu.CompilerParams(
            dimension_semantics=("parallel",)),
    )(xh)
    return jnp.transpose(out.reshape(n, ho, wo, c), (0, 3, 1, 2))


def kernel(x0, x1, x2, x3, p0_w1, p0_b1, p0_gamma, p0_beta, p0_mean, p0_var,
           p0_w2, p0_b2, p1_w, p1_b):
    n, c, h, w = x0.shape

    y0 = _deconv4x_nhwc(_row_view_bf16(x0), p0_w1, p0_b1, p0_gamma, p0_beta,
                        p0_mean, p0_var, p0_w2, p0_b2, n=n, h=h, w=w)
    out0 = jnp.transpose(y0, (0, 3, 1, 2))

    y1 = _deconv2x_nhwc(_row_view_bf16(x1), p1_w, p1_b, n=n, h=h, w=w)
    out1 = jnp.transpose(y1, (0, 3, 1, 2))

    out2 = x2

    out3 = _maxpool2x2(x3)

    return (out0, out1, out2, out3)
